# baseline (speedup 1.0000x reference)
import jax
import jax.numpy as jnp
from jax import lax
from jax.experimental import pallas as pl
from jax.experimental.pallas import tpu as pltpu
from jax.experimental.pallas import tpu_sc as plsc

_C, _F, _O, _B = 6, 1000, 256, 16384
_N = _C * _F * _O
_NC, _NS, _L = 2, 16, 16
_NW = _NC * _NS
_BPW = _B // _NW
_ICH = 128
_NL = _BPW // _ICH
_NT = _BPW * 3 // _ICH
_NR = _BPW * 4 // _ICH


def _pose_body(it_hbm, ir_hbm, ot_hbm, oth_hbm, cam_hbm, frm_hbm, obj_hbm,
               otr_hbm, orot_hbm,
               cam_v, frm_v, obj_v, lin_v, lin2_v, idxt_v, idxr_v,
               ta_v, tb_v, q_v, th_v, otr_v, oq_v, sem):
    wid = lax.axis_index("s") * _NC + lax.axis_index("c")
    base = wid * _BPW
    lane = lax.iota(jnp.int32, _L)

    pltpu.sync_copy(cam_hbm.at[pl.ds(base, _BPW)], cam_v)
    pltpu.sync_copy(frm_hbm.at[pl.ds(base, _BPW)], frm_v)
    pltpu.sync_copy(obj_hbm.at[pl.ds(base, _BPW)], obj_v)

    for k in range(_NL):
        def lin_body(j, s, k=k):
            c16 = cam_v[pl.ds(s, _L)]
            f16 = frm_v[pl.ds(s, _L)]
            o16 = obj_v[pl.ds(s, _L)]
            lin = (c16 * _F + f16) * _O + o16
            lin_v[pl.ds(s, _L)] = lin
            lin2_v[k, pl.ds(s - k * _ICH, _L)] = lin * 4
            return s + _L
        lax.fori_loop(0, _ICH // _L, lin_body, k * _ICH)

    three = jnp.full((_L,), 3, jnp.int32)
    for k in range(_NT):
        def idxt_body(j, f, k=k):
            b = lax.div(f, three)
            r = f - b * 3
            idxt_v[k, pl.ds(j * _L, _L)] = plsc.load_gather(lin_v, [b]) * 4 + r
            return f + _L
        lax.fori_loop(0, _ICH // _L, idxt_body, k * _ICH + lane)

    for k in range(_NR):
        def idxr_body(j, f, k=k):
            b = lax.shift_right_logical(f, 2)
            r = lax.bitwise_and(f, 3)
            idxr_v[k, pl.ds(j * _L, _L)] = plsc.load_gather(lin_v, [b]) * 4 + r
            return f + _L
        lax.fori_loop(0, _ICH // _L, idxr_body, k * _ICH + lane)

    copies = []
    for k in range(_NT):
        d = pl.ds(k * _ICH, _ICH)
        copies.append(pltpu.async_copy(it_hbm.at[idxt_v.at[k]], ta_v.at[d], sem))
        copies.append(pltpu.async_copy(ot_hbm.at[idxt_v.at[k]], tb_v.at[d], sem))
    for k in range(_NR):
        d = pl.ds(k * _ICH, _ICH)
        copies.append(pltpu.async_copy(ir_hbm.at[idxr_v.at[k]], q_v.at[d], sem))
    for k in range(_NL):
        d = pl.ds(k * _ICH, _ICH)
        copies.append(pltpu.async_copy(oth_hbm.at[lin2_v.at[k]], th_v.at[d], sem))
    for cp in copies:
        cp.wait()

    def tr_body(j, s):
        otr_v[pl.ds(s, _L)] = ta_v[pl.ds(s, _L)] + tb_v[pl.ds(s, _L)]
        return s + _L
    lax.fori_loop(0, _BPW * 3 // _L, tr_body, 0)

    rev = lane + 3 - 2 * lax.bitwise_and(lane, 3)
    quart = lax.shift_right_logical(lane, 2)
    sgn = jnp.where(lax.bitwise_and(lane, 1) == 1,
                    jnp.float32(1.0), jnp.float32(-1.0))

    def rot_body(j, carry):
        frev, rowq, s16 = carry
        a = q_v[pl.ds(s16, _L)]
        ar = plsc.load_gather(q_v, [frev])
        th = plsc.load_gather(th_v, [rowq])
        h = th * jnp.float32(0.5)
        h2 = h * h
        c = jnp.float32(1.0) + h2 * (
            jnp.float32(-1 / 2) + h2 * (
                jnp.float32(1 / 24) + h2 * (
                    jnp.float32(-1 / 720) + h2 * jnp.float32(1 / 40320))))
        s = h * (jnp.float32(1.0) + h2 * (
            jnp.float32(-1 / 6) + h2 * (
                jnp.float32(1 / 120) + h2 * (
                    jnp.float32(-1 / 5040) + h2 * jnp.float32(1 / 362880)))))
        o = a * c + ar * s * sgn
        oq_v[pl.ds(s16, _L)] = o
        return (frev + _L, rowq + 4, s16 + _L)
    lax.fori_loop(0, _BPW * 4 // _L, rot_body, (rev, quart, 0))

    pltpu.sync_copy(otr_v, otr_hbm.at[pl.ds(base * 3, _BPW * 3)])
    pltpu.sync_copy(oq_v, orot_hbm.at[pl.ds(base * 4, _BPW * 4)])


_pose_call = pl.kernel(
    _pose_body,
    mesh=plsc.VectorSubcoreMesh(core_axis_name="c", subcore_axis_name="s"),
    compiler_params=pltpu.CompilerParams(
        use_tc_tiling_on_sc=False, needs_layout_passes=False),
    out_type=(
        jax.ShapeDtypeStruct((_B * 3,), jnp.float32),
        jax.ShapeDtypeStruct((_B * 4,), jnp.float32),
    ),
    scratch_types=[
        pltpu.VMEM((_BPW,), jnp.int32),
        pltpu.VMEM((_BPW,), jnp.int32),
        pltpu.VMEM((_BPW,), jnp.int32),
        pltpu.VMEM((_BPW,), jnp.int32),
        pltpu.VMEM((_NL, _ICH), jnp.int32),
        pltpu.VMEM((_NT, _ICH), jnp.int32),
        pltpu.VMEM((_NR, _ICH), jnp.int32),
        pltpu.VMEM((_BPW * 3,), jnp.float32),
        pltpu.VMEM((_BPW * 3,), jnp.float32),
        pltpu.VMEM((_BPW * 4,), jnp.float32),
        pltpu.VMEM((_BPW,), jnp.float32),
        pltpu.VMEM((_BPW * 3,), jnp.float32),
        pltpu.VMEM((_BPW * 4,), jnp.float32),
        pltpu.SemaphoreType.DMA,
    ],
)


@jax.jit
def kernel(input_trans, input_rots, opt_trans, opt_rots, cam, frame_idx, obj_id):
    z1 = jnp.zeros((_C, _F, _O, 1), jnp.float32)
    z3 = jnp.zeros((_C, _F, _O, 3), jnp.float32)
    it = jnp.concatenate([input_trans, z1], axis=-1).reshape(-1)
    ot = jnp.concatenate([opt_trans, z1], axis=-1).reshape(-1)
    oth = jnp.concatenate([opt_rots, z3], axis=-1).reshape(-1)
    ir = input_rots.reshape(-1)
    cam = cam.astype(jnp.int32)
    frm = frame_idx.astype(jnp.int32)
    obj = obj_id.astype(jnp.int32)
    tr, rot = _pose_call(it, ir, ot, oth, cam, frm, obj)
    return tr.reshape(_B, 3), rot.reshape(_B, 4)


# speedup vs baseline: 200.1154x; 200.1154x over previous
import jax
import jax.numpy as jnp
from jax import lax
from jax.experimental import pallas as pl
from jax.experimental.pallas import tpu as pltpu
from jax.experimental.pallas import tpu_sc as plsc

_C, _F, _O, _B = 6, 1000, 256, 16384
_N = _C * _F * _O
_NC, _NS, _L = 2, 16, 16
_NW = _NC * _NS
_BPW = _B // _NW
_ICH = 128
_NL = _BPW // _ICH
_NT = _BPW * 3 // _ICH
_NR = _BPW * 4 // _ICH

_TD = 256000


def _pose_body(it_hbm, ir_hbm, ot_hbm, oth_hbm, cam_hbm, frm_hbm, obj_hbm,
               otr_hbm, orot_hbm,
               cam_v, frm_v, obj_v, p_v, q_v, lin2_v, idxt_v, idxr_v,
               ta_v, tb_v, qr_v, th_v, otr_v, oq_v, sem):
    wid = lax.axis_index("s") * _NC + lax.axis_index("c")
    base = wid * _BPW
    lane = lax.iota(jnp.int32, _L)

    pltpu.sync_copy(cam_hbm.at[pl.ds(base, _BPW)], cam_v)
    pltpu.sync_copy(frm_hbm.at[pl.ds(base, _BPW)], frm_v)
    pltpu.sync_copy(obj_hbm.at[pl.ds(base, _BPW)], obj_v)

    for k in range(_NL):
        def lin_body(j, s, k=k):
            c16 = cam_v[pl.ds(s, _L)]
            f16 = frm_v[pl.ds(s, _L)]
            o16 = obj_v[pl.ds(s, _L)]
            fhi = lax.shift_right_logical(f16, 3)
            flo = lax.bitwise_and(f16, 7)
            ohi = lax.shift_right_logical(o16, 7)
            olo = lax.bitwise_and(o16, 127)
            cf = c16 * _F + f16
            p_v[pl.ds(s, _L)] = (c16 * 3 * _TD + fhi * 2048 + ohi * 1024
                                 + flo * 128 + olo)
            q_v[pl.ds(s, _L)] = cf * 1024 + ohi * 512 + olo
            lin2_v[k, pl.ds(s - k * _ICH, _L)] = cf * _O + o16
            return s + _L
        lax.fori_loop(0, _ICH // _L, lin_body, k * _ICH)

    three = jnp.full((_L,), 3, jnp.int32)
    for k in range(_NT):
        def idxt_body(j, f, k=k):
            b = lax.div(f, three)
            r = f - b * 3
            idxt_v[k, pl.ds(j * _L, _L)] = (
                plsc.load_gather(p_v, [b]) + r * _TD)
            return f + _L
        lax.fori_loop(0, _ICH // _L, idxt_body, k * _ICH + lane)

    for k in range(_NR):
        def idxr_body(j, f, k=k):
            b = lax.shift_right_logical(f, 2)
            r = lax.bitwise_and(f, 3)
            idxr_v[k, pl.ds(j * _L, _L)] = (
                plsc.load_gather(q_v, [b]) + r * 128)
            return f + _L
        lax.fori_loop(0, _ICH // _L, idxr_body, k * _ICH + lane)

    copies = []
    for k in range(_NT):
        d = pl.ds(k * _ICH, _ICH)
        copies.append(pltpu.async_copy(it_hbm.at[idxt_v.at[k]], ta_v.at[d], sem))
        copies.append(pltpu.async_copy(ot_hbm.at[idxt_v.at[k]], tb_v.at[d], sem))
    for k in range(_NR):
        d = pl.ds(k * _ICH, _ICH)
        copies.append(pltpu.async_copy(ir_hbm.at[idxr_v.at[k]], qr_v.at[d], sem))
    for k in range(_NL):
        d = pl.ds(k * _ICH, _ICH)
        copies.append(pltpu.async_copy(oth_hbm.at[lin2_v.at[k]], th_v.at[d], sem))
    for cp in copies:
        cp.wait()

    def tr_body(j, s):
        otr_v[pl.ds(s, _L)] = ta_v[pl.ds(s, _L)] + tb_v[pl.ds(s, _L)]
        return s + _L
    lax.fori_loop(0, _BPW * 3 // _L, tr_body, 0)

    rev = lane + 3 - 2 * lax.bitwise_and(lane, 3)
    quart = lax.shift_right_logical(lane, 2)
    sgn = jnp.where(lax.bitwise_and(lane, 1) == 1,
                    jnp.float32(1.0), jnp.float32(-1.0))

    def rot_body(j, carry):
        frev, rowq, s16 = carry
        a = qr_v[pl.ds(s16, _L)]
        ar = plsc.load_gather(qr_v, [frev])
        th = plsc.load_gather(th_v, [rowq])
        h = th * jnp.float32(0.5)
        h2 = h * h
        c = jnp.float32(1.0) + h2 * (
            jnp.float32(-1 / 2) + h2 * (
                jnp.float32(1 / 24) + h2 * (
                    jnp.float32(-1 / 720) + h2 * jnp.float32(1 / 40320))))
        s = h * (jnp.float32(1.0) + h2 * (
            jnp.float32(-1 / 6) + h2 * (
                jnp.float32(1 / 120) + h2 * (
                    jnp.float32(-1 / 5040) + h2 * jnp.float32(1 / 362880)))))
        o = a * c + ar * s * sgn
        oq_v[pl.ds(s16, _L)] = o
        return (frev + _L, rowq + 4, s16 + _L)
    lax.fori_loop(0, _BPW * 4 // _L, rot_body, (rev, quart, 0))

    pltpu.sync_copy(otr_v, otr_hbm.at[pl.ds(base * 3, _BPW * 3)])
    pltpu.sync_copy(oq_v, orot_hbm.at[pl.ds(base * 4, _BPW * 4)])


_pose_call = pl.kernel(
    _pose_body,
    mesh=plsc.VectorSubcoreMesh(core_axis_name="c", subcore_axis_name="s"),
    compiler_params=pltpu.CompilerParams(
        use_tc_tiling_on_sc=False, needs_layout_passes=False),
    out_type=(
        jax.ShapeDtypeStruct((_B * 3,), jnp.float32),
        jax.ShapeDtypeStruct((_B * 4,), jnp.float32),
    ),
    scratch_types=[
        pltpu.VMEM((_BPW,), jnp.int32),
        pltpu.VMEM((_BPW,), jnp.int32),
        pltpu.VMEM((_BPW,), jnp.int32),
        pltpu.VMEM((_BPW,), jnp.int32),
        pltpu.VMEM((_BPW,), jnp.int32),
        pltpu.VMEM((_NL, _ICH), jnp.int32),
        pltpu.VMEM((_NT, _ICH), jnp.int32),
        pltpu.VMEM((_NR, _ICH), jnp.int32),
        pltpu.VMEM((_BPW * 3,), jnp.float32),
        pltpu.VMEM((_BPW * 3,), jnp.float32),
        pltpu.VMEM((_BPW * 4,), jnp.float32),
        pltpu.VMEM((_BPW,), jnp.float32),
        pltpu.VMEM((_BPW * 3,), jnp.float32),
        pltpu.VMEM((_BPW * 4,), jnp.float32),
        pltpu.SemaphoreType.DMA,
    ],
)


def _trans_view(t):
    v = t.transpose(0, 3, 1, 2).reshape(_C, 3, _F // 8, 8, 2, 128)
    return v.transpose(0, 1, 2, 4, 3, 5).reshape(-1)


def _rots_view(t):
    return t.reshape(_C, _F, 2, 128, 4).transpose(0, 1, 2, 4, 3).reshape(-1)


@jax.jit
def kernel(input_trans, input_rots, opt_trans, opt_rots, cam, frame_idx, obj_id):
    it = _trans_view(input_trans)
    ot = _trans_view(opt_trans)
    ir = _rots_view(input_rots)
    oth = opt_rots.reshape(-1)
    cam = cam.astype(jnp.int32)
    frm = frame_idx.astype(jnp.int32)
    obj = obj_id.astype(jnp.int32)
    tr, rot = _pose_call(it, ir, ot, oth, cam, frm, obj)
    return tr.reshape(_B, 3), rot.reshape(_B, 4)


# speedup vs baseline: 438.5941x; 2.1917x over previous
import jax
import jax.numpy as jnp
from jax import lax
from jax.experimental import pallas as pl
from jax.experimental.pallas import tpu as pltpu
from jax.experimental.pallas import tpu_sc as plsc

_C, _F, _O, _B = 6, 1000, 256, 16384
_NC, _NS, _L = 2, 16, 16
_NW = _NC * _NS
_BPW = _B // _NW
_ICH = 128
_NBT = _BPW // _ICH
_JPB = _ICH // _L

_TD = 256000


def _pose_body(it_hbm, ir_hbm, ot_hbm, oth_hbm, cam_hbm, frm_hbm, obj_hbm,
               otr_hbm, orot_hbm,
               cam_v, frm_v, obj_v, p_v, lin2_v, idxt_v, idxr_v,
               ta_v, tb_v, qr_v, th_v, otr_v, oq_v, sem):
    wid = lax.axis_index("s") * _NC + lax.axis_index("c")
    base = wid * _BPW

    pltpu.sync_copy(cam_hbm.at[pl.ds(base, _BPW)], cam_v)
    pltpu.sync_copy(frm_hbm.at[pl.ds(base, _BPW)], frm_v)
    pltpu.sync_copy(obj_hbm.at[pl.ds(base, _BPW)], obj_v)

    for bt in range(_NBT):
        def lin_body(j, s, bt=bt):
            c16 = cam_v[pl.ds(s, _L)]
            f16 = frm_v[pl.ds(s, _L)]
            o16 = obj_v[pl.ds(s, _L)]
            fhi = lax.shift_right_logical(f16, 3)
            flo = lax.bitwise_and(f16, 7)
            ohi = lax.shift_right_logical(o16, 7)
            olo = lax.bitwise_and(o16, 127)
            cf = c16 * _F + f16
            p = c16 * (3 * _TD) + fhi * 2048 + ohi * 1024 + flo * 128 + olo
            q = cf * 1024 + ohi * 512 + olo
            p_v[pl.ds(s, _L)] = p
            sj = pl.ds(s - bt * _ICH, _L)
            for d in range(3):
                idxt_v[bt * 3 + d, sj] = p + d * _TD
            for d in range(4):
                idxr_v[bt * 4 + d, sj] = q + d * 128
            lin2_v[bt, sj] = cf * _O + o16
            return s + _L
        lax.fori_loop(0, _JPB, lin_body, bt * _ICH)

    copies = []
    for bt in range(_NBT):
        for d in range(3):
            dst = pl.ds(bt * 512 + d * _ICH, _ICH)
            idx = idxt_v.at[bt * 3 + d]
            copies.append(pltpu.async_copy(it_hbm.at[idx], ta_v.at[dst], sem))
            copies.append(pltpu.async_copy(ot_hbm.at[idx], tb_v.at[dst], sem))
        for d in range(4):
            dst = pl.ds(bt * 512 + d * _ICH, _ICH)
            idx = idxr_v.at[bt * 4 + d]
            copies.append(pltpu.async_copy(ir_hbm.at[idx], qr_v.at[dst], sem))
        dst = pl.ds(bt * _ICH, _ICH)
        copies.append(pltpu.async_copy(oth_hbm.at[lin2_v.at[bt]], th_v.at[dst], sem))
    for cp in copies:
        cp.wait()

    zero16 = jnp.zeros((_L,), jnp.float32)

    for bt in range(_NBT):
        def tr_body(j, s, bt=bt):
            for d in range(3):
                sl = pl.ds(s + d * _ICH, _L)
                otr_v[sl] = ta_v[sl] + tb_v[sl]
            otr_v[pl.ds(s + 3 * _ICH, _L)] = zero16
            return s + _L
        lax.fori_loop(0, _JPB, tr_body, bt * 512)

    for bt in range(_NBT):
        def rot_body(j, carry, bt=bt):
            sth, s = carry
            th = th_v[pl.ds(sth, _L)]
            h = th * jnp.float32(0.5)
            h2 = h * h
            c = jnp.float32(1.0) + h2 * (
                jnp.float32(-1 / 2) + h2 * (
                    jnp.float32(1 / 24) + h2 * (
                        jnp.float32(-1 / 720) + h2 * jnp.float32(1 / 40320))))
            sn = h * (jnp.float32(1.0) + h2 * (
                jnp.float32(-1 / 6) + h2 * (
                    jnp.float32(1 / 120) + h2 * (
                        jnp.float32(-1 / 5040) + h2 * jnp.float32(1 / 362880)))))
            a0 = qr_v[pl.ds(s, _L)]
            a1 = qr_v[pl.ds(s + _ICH, _L)]
            a2 = qr_v[pl.ds(s + 2 * _ICH, _L)]
            a3 = qr_v[pl.ds(s + 3 * _ICH, _L)]
            oq_v[pl.ds(s, _L)] = a0 * c - a3 * sn
            oq_v[pl.ds(s + _ICH, _L)] = a1 * c + a2 * sn
            oq_v[pl.ds(s + 2 * _ICH, _L)] = a2 * c - a1 * sn
            oq_v[pl.ds(s + 3 * _ICH, _L)] = a3 * c + a0 * sn
            return (sth + _L, s + _L)
        lax.fori_loop(0, _JPB, rot_body, (bt * _ICH, bt * 512))

    pltpu.sync_copy(otr_v, otr_hbm.at[pl.ds(wid * 2048, 2048)])
    pltpu.sync_copy(oq_v, orot_hbm.at[pl.ds(wid * 2048, 2048)])


_pose_call = pl.kernel(
    _pose_body,
    mesh=plsc.VectorSubcoreMesh(core_axis_name="c", subcore_axis_name="s"),
    compiler_params=pltpu.CompilerParams(
        use_tc_tiling_on_sc=False, needs_layout_passes=False),
    out_type=(
        jax.ShapeDtypeStruct((_B * 4,), jnp.float32),
        jax.ShapeDtypeStruct((_B * 4,), jnp.float32),
    ),
    scratch_types=[
        pltpu.VMEM((_BPW,), jnp.int32),
        pltpu.VMEM((_BPW,), jnp.int32),
        pltpu.VMEM((_BPW,), jnp.int32),
        pltpu.VMEM((_BPW,), jnp.int32),
        pltpu.VMEM((_NBT, _ICH), jnp.int32),
        pltpu.VMEM((_NBT * 3, _ICH), jnp.int32),
        pltpu.VMEM((_NBT * 4, _ICH), jnp.int32),
        pltpu.VMEM((_BPW * 4,), jnp.float32),
        pltpu.VMEM((_BPW * 4,), jnp.float32),
        pltpu.VMEM((_BPW * 4,), jnp.float32),
        pltpu.VMEM((_BPW,), jnp.float32),
        pltpu.VMEM((_BPW * 4,), jnp.float32),
        pltpu.VMEM((_BPW * 4,), jnp.float32),
        pltpu.SemaphoreType.DMA,
    ],
)


def _trans_view(t):
    v = t.transpose(0, 3, 1, 2).reshape(_C, 3, _F // 8, 8, 2, 128)
    return v.transpose(0, 1, 2, 4, 3, 5).reshape(-1)


def _rots_view(t):
    return t.reshape(_C, _F, 2, 128, 4).transpose(0, 1, 2, 4, 3).reshape(-1)


def _planar_out(flat):
    return flat.reshape(_B // 128, 4, 128).transpose(0, 2, 1).reshape(_B, 4)


@jax.jit
def kernel(input_trans, input_rots, opt_trans, opt_rots, cam, frame_idx, obj_id):
    it = _trans_view(input_trans)
    ot = _trans_view(opt_trans)
    ir = _rots_view(input_rots)
    oth = opt_rots.reshape(-1)
    cam = cam.astype(jnp.int32)
    frm = frame_idx.astype(jnp.int32)
    obj = obj_id.astype(jnp.int32)
    tr, rot = _pose_call(it, ir, ot, oth, cam, frm, obj)
    return _planar_out(tr)[:, :3], _planar_out(rot)


# speedup vs baseline: 440.0525x; 1.0033x over previous
import jax
import jax.numpy as jnp
from jax import lax
from jax.experimental import pallas as pl
from jax.experimental.pallas import tpu as pltpu
from jax.experimental.pallas import tpu_sc as plsc

_C, _F, _O, _B = 6, 1000, 256, 16384
_NC, _NS, _L = 2, 16, 16
_NW = _NC * _NS
_BPW = _B // _NW
_ICH = 128
_NBT = _BPW // _ICH
_JPB = _ICH // _L

_TD = 256000


def _pose_body(it_hbm, ir_hbm, ot_hbm, oth_hbm, cam_hbm, frm_hbm, obj_hbm,
               otr_hbm, orot_hbm,
               cam_v, frm_v, obj_v, lin2_v, idxt_v, idxr_v,
               ta_v, tb_v, qr_v, th_v, otr_v, oq_v, sem, osem):
    wid = lax.axis_index("s") * _NC + lax.axis_index("c")
    base = wid * _BPW

    pltpu.sync_copy(cam_hbm.at[pl.ds(base, _BPW)], cam_v)
    pltpu.sync_copy(frm_hbm.at[pl.ds(base, _BPW)], frm_v)
    pltpu.sync_copy(obj_hbm.at[pl.ds(base, _BPW)], obj_v)

    for bt in range(_NBT):
        def lin_body(j, s, bt=bt):
            c16 = cam_v[pl.ds(s, _L)]
            f16 = frm_v[pl.ds(s, _L)]
            o16 = obj_v[pl.ds(s, _L)]
            fhi = lax.shift_right_logical(f16, 3)
            flo = lax.bitwise_and(f16, 7)
            ohi = lax.shift_right_logical(o16, 7)
            olo = lax.bitwise_and(o16, 127)
            cf = c16 * _F + f16
            p = c16 * (3 * _TD) + fhi * 2048 + ohi * 1024 + flo * 128 + olo
            q = cf * 1024 + ohi * 512 + olo
            sj = pl.ds(s - bt * _ICH, _L)
            for d in range(3):
                idxt_v[bt * 3 + d, sj] = p + d * _TD
            for d in range(4):
                idxr_v[bt * 4 + d, sj] = q + d * 128
            lin2_v[bt, sj] = cf * _O + o16
            return s + _L
        lax.fori_loop(0, _JPB, lin_body, bt * _ICH)

    copies = []
    for bt in range(_NBT):
        blk = []
        for d in range(3):
            dst = pl.ds(bt * 512 + d * _ICH, _ICH)
            idx = idxt_v.at[bt * 3 + d]
            blk.append(pltpu.async_copy(it_hbm.at[idx], ta_v.at[dst], sem))
            blk.append(pltpu.async_copy(ot_hbm.at[idx], tb_v.at[dst], sem))
        for d in range(4):
            dst = pl.ds(bt * 512 + d * _ICH, _ICH)
            idx = idxr_v.at[bt * 4 + d]
            blk.append(pltpu.async_copy(ir_hbm.at[idx], qr_v.at[dst], sem))
        dst = pl.ds(bt * _ICH, _ICH)
        blk.append(pltpu.async_copy(oth_hbm.at[lin2_v.at[bt]], th_v.at[dst], sem))
        copies.append(blk)

    zero16 = jnp.zeros((_L,), jnp.float32)

    for bt in range(_NBT):
        for cp in copies[bt]:
            cp.wait()

        def tr_body(j, s, bt=bt):
            for d in range(3):
                sl = pl.ds(s + d * _ICH, _L)
                otr_v[sl] = ta_v[sl] + tb_v[sl]
            otr_v[pl.ds(s + 3 * _ICH, _L)] = zero16
            return s + _L
        lax.fori_loop(0, _JPB, tr_body, bt * 512)

        def rot_body(j, carry, bt=bt):
            sth, s = carry
            th = th_v[pl.ds(sth, _L)]
            h = th * jnp.float32(0.5)
            h2 = h * h
            c = jnp.float32(1.0) + h2 * (
                jnp.float32(-1 / 2) + h2 * (
                    jnp.float32(1 / 24) + h2 * (
                        jnp.float32(-1 / 720) + h2 * jnp.float32(1 / 40320))))
            sn = h * (jnp.float32(1.0) + h2 * (
                jnp.float32(-1 / 6) + h2 * (
                    jnp.float32(1 / 120) + h2 * (
                        jnp.float32(-1 / 5040) + h2 * jnp.float32(1 / 362880)))))
            a0 = qr_v[pl.ds(s, _L)]
            a1 = qr_v[pl.ds(s + _ICH, _L)]
            a2 = qr_v[pl.ds(s + 2 * _ICH, _L)]
            a3 = qr_v[pl.ds(s + 3 * _ICH, _L)]
            oq_v[pl.ds(s, _L)] = a0 * c - a3 * sn
            oq_v[pl.ds(s + _ICH, _L)] = a1 * c + a2 * sn
            oq_v[pl.ds(s + 2 * _ICH, _L)] = a2 * c - a1 * sn
            oq_v[pl.ds(s + 3 * _ICH, _L)] = a3 * c + a0 * sn
            return (sth + _L, s + _L)
        lax.fori_loop(0, _JPB, rot_body, (bt * _ICH, bt * 512))

    o1 = pltpu.async_copy(otr_v, otr_hbm.at[pl.ds(wid * 2048, 2048)], osem)
    o2 = pltpu.async_copy(oq_v, orot_hbm.at[pl.ds(wid * 2048, 2048)], osem)
    o1.wait()
    o2.wait()


_pose_call = pl.kernel(
    _pose_body,
    mesh=plsc.VectorSubcoreMesh(core_axis_name="c", subcore_axis_name="s"),
    compiler_params=pltpu.CompilerParams(
        use_tc_tiling_on_sc=False, needs_layout_passes=False),
    out_type=(
        jax.ShapeDtypeStruct((_B * 4,), jnp.float32),
        jax.ShapeDtypeStruct((_B * 4,), jnp.float32),
    ),
    scratch_types=[
        pltpu.VMEM((_BPW,), jnp.int32),
        pltpu.VMEM((_BPW,), jnp.int32),
        pltpu.VMEM((_BPW,), jnp.int32),
        pltpu.VMEM((_NBT, _ICH), jnp.int32),
        pltpu.VMEM((_NBT * 3, _ICH), jnp.int32),
        pltpu.VMEM((_NBT * 4, _ICH), jnp.int32),
        pltpu.VMEM((_BPW * 4,), jnp.float32),
        pltpu.VMEM((_BPW * 4,), jnp.float32),
        pltpu.VMEM((_BPW * 4,), jnp.float32),
        pltpu.VMEM((_BPW,), jnp.float32),
        pltpu.VMEM((_BPW * 4,), jnp.float32),
        pltpu.VMEM((_BPW * 4,), jnp.float32),
        pltpu.SemaphoreType.DMA,
        pltpu.SemaphoreType.DMA,
    ],
)


def _trans_view(t):
    v = t.transpose(0, 3, 1, 2).reshape(_C, 3, _F // 8, 8, 2, 128)
    return v.transpose(0, 1, 2, 4, 3, 5).reshape(-1)


def _rots_view(t):
    return t.reshape(_C, _F, 2, 128, 4).transpose(0, 1, 2, 4, 3).reshape(-1)


def _planar_out(flat):
    return flat.reshape(_B // 128, 4, 128).transpose(0, 2, 1).reshape(_B, 4)


@jax.jit
def kernel(input_trans, input_rots, opt_trans, opt_rots, cam, frame_idx, obj_id):
    it = _trans_view(input_trans)
    ot = _trans_view(opt_trans)
    ir = _rots_view(input_rots)
    oth = opt_rots.reshape(-1)
    cam = cam.astype(jnp.int32)
    frm = frame_idx.astype(jnp.int32)
    obj = obj_id.astype(jnp.int32)
    tr, rot = _pose_call(it, ir, ot, oth, cam, frm, obj)
    return _planar_out(tr)[:, :3], _planar_out(rot)


# speedup vs baseline: 458.6569x; 1.0423x over previous
import jax
import jax.numpy as jnp
from jax import lax
from jax.experimental import pallas as pl
from jax.experimental.pallas import tpu as pltpu
from jax.experimental.pallas import tpu_sc as plsc

_C, _F, _O, _B = 6, 1000, 256, 16384
_NC, _NS, _L = 2, 16, 16
_NW = _NC * _NS
_BPW = _B // _NW
_NBT = _BPW // 128
_JPB = 128 // _L

_TD = 256000


def _pose_body(it_hbm, ir_hbm, ot_hbm, oth_hbm, cam_hbm, frm_hbm, obj_hbm,
               otr_hbm, orot_hbm,
               cam_v, frm_v, obj_v, lin_v, idxt_v, idxr_v,
               ta_v, tb_v, qr_v, th_v, otr_v, oq_v, sem, osem):
    wid = lax.axis_index("s") * _NC + lax.axis_index("c")
    base = wid * _BPW

    s1 = pltpu.async_copy(cam_hbm.at[pl.ds(base, _BPW)], cam_v, sem)
    s2 = pltpu.async_copy(frm_hbm.at[pl.ds(base, _BPW)], frm_v, sem)
    s3 = pltpu.async_copy(obj_hbm.at[pl.ds(base, _BPW)], obj_v, sem)
    s1.wait()
    s2.wait()
    s3.wait()

    for bt in range(_NBT):
        def lin_body(j, sj, bt=bt):
            s = pl.ds(bt * 128 + sj, _L)
            c16 = cam_v[s]
            f16 = frm_v[s]
            o16 = obj_v[s]
            fhi = lax.shift_right_logical(f16, 3)
            flo = lax.bitwise_and(f16, 7)
            ohi = lax.shift_right_logical(o16, 7)
            olo = lax.bitwise_and(o16, 127)
            cf = c16 * _F + f16
            p = c16 * (3 * _TD) + fhi * 2048 + ohi * 1024 + flo * 128 + olo
            q = cf * 1024 + ohi * 512 + olo
            for d in range(3):
                idxt_v[pl.ds(bt * 384 + d * 128 + sj, _L)] = p + d * _TD
            for d in range(4):
                idxr_v[pl.ds(bt * 512 + d * 128 + sj, _L)] = q + d * 128
            lin_v[s] = cf * _O + o16
            return sj + _L
        lax.fori_loop(0, _JPB, lin_body, 0)

    copies = [
        pltpu.async_copy(it_hbm.at[idxt_v], ta_v, sem),
        pltpu.async_copy(ot_hbm.at[idxt_v], tb_v, sem),
        pltpu.async_copy(ir_hbm.at[idxr_v], qr_v, sem),
        pltpu.async_copy(oth_hbm.at[lin_v], th_v, sem),
    ]
    for cp in copies:
        cp.wait()

    zero16 = jnp.zeros((_L,), jnp.float32)

    for bt in range(_NBT):
        def tr_body(j, sj, bt=bt):
            for d in range(3):
                sl = pl.ds(bt * 384 + d * 128 + sj, _L)
                otr_v[pl.ds(bt * 512 + d * 128 + sj, _L)] = ta_v[sl] + tb_v[sl]
            otr_v[pl.ds(bt * 512 + 3 * 128 + sj, _L)] = zero16
            return sj + _L
        lax.fori_loop(0, _JPB, tr_body, 0)

        def rot_body(j, carry, bt=bt):
            sth, s = carry
            th = th_v[pl.ds(sth, _L)]
            h = th * jnp.float32(0.5)
            h2 = h * h
            c = jnp.float32(1.0) + h2 * (
                jnp.float32(-1 / 2) + h2 * (
                    jnp.float32(1 / 24) + h2 * (
                        jnp.float32(-1 / 720) + h2 * jnp.float32(1 / 40320))))
            sn = h * (jnp.float32(1.0) + h2 * (
                jnp.float32(-1 / 6) + h2 * (
                    jnp.float32(1 / 120) + h2 * (
                        jnp.float32(-1 / 5040) + h2 * jnp.float32(1 / 362880)))))
            a0 = qr_v[pl.ds(s, _L)]
            a1 = qr_v[pl.ds(s + 128, _L)]
            a2 = qr_v[pl.ds(s + 2 * 128, _L)]
            a3 = qr_v[pl.ds(s + 3 * 128, _L)]
            oq_v[pl.ds(s, _L)] = a0 * c - a3 * sn
            oq_v[pl.ds(s + 128, _L)] = a1 * c + a2 * sn
            oq_v[pl.ds(s + 2 * 128, _L)] = a2 * c - a1 * sn
            oq_v[pl.ds(s + 3 * 128, _L)] = a3 * c + a0 * sn
            return (sth + _L, s + _L)
        lax.fori_loop(0, _JPB, rot_body, (bt * 128, bt * 512))

    o1 = pltpu.async_copy(otr_v, otr_hbm.at[pl.ds(wid * 2048, 2048)], osem)
    o2 = pltpu.async_copy(oq_v, orot_hbm.at[pl.ds(wid * 2048, 2048)], osem)
    o1.wait()
    o2.wait()


_pose_call = pl.kernel(
    _pose_body,
    mesh=plsc.VectorSubcoreMesh(core_axis_name="c", subcore_axis_name="s"),
    compiler_params=pltpu.CompilerParams(
        use_tc_tiling_on_sc=False, needs_layout_passes=False),
    out_type=(
        jax.ShapeDtypeStruct((_B * 4,), jnp.float32),
        jax.ShapeDtypeStruct((_B * 4,), jnp.float32),
    ),
    scratch_types=[
        pltpu.VMEM((_BPW,), jnp.int32),
        pltpu.VMEM((_BPW,), jnp.int32),
        pltpu.VMEM((_BPW,), jnp.int32),
        pltpu.VMEM((_BPW,), jnp.int32),
        pltpu.VMEM((_BPW * 3,), jnp.int32),
        pltpu.VMEM((_BPW * 4,), jnp.int32),
        pltpu.VMEM((_BPW * 3,), jnp.float32),
        pltpu.VMEM((_BPW * 3,), jnp.float32),
        pltpu.VMEM((_BPW * 4,), jnp.float32),
        pltpu.VMEM((_BPW,), jnp.float32),
        pltpu.VMEM((_BPW * 4,), jnp.float32),
        pltpu.VMEM((_BPW * 4,), jnp.float32),
        pltpu.SemaphoreType.DMA,
        pltpu.SemaphoreType.DMA,
    ],
)


def _trans_view(t):
    v = t.transpose(0, 3, 1, 2).reshape(_C, 3, _F // 8, 8, 2, 128)
    return v.transpose(0, 1, 2, 4, 3, 5).reshape(-1)


def _rots_view(t):
    return t.reshape(_C, _F, 2, 128, 4).transpose(0, 1, 2, 4, 3).reshape(-1)


def _planar_out(flat):
    return flat.reshape(_B // 128, 4, 128).transpose(0, 2, 1).reshape(_B, 4)


@jax.jit
def kernel(input_trans, input_rots, opt_trans, opt_rots, cam, frame_idx, obj_id):
    it = _trans_view(input_trans)
    ot = _trans_view(opt_trans)
    ir = _rots_view(input_rots)
    oth = opt_rots.reshape(-1)
    cam = cam.astype(jnp.int32)
    frm = frame_idx.astype(jnp.int32)
    obj = obj_id.astype(jnp.int32)
    tr, rot = _pose_call(it, ir, ot, oth, cam, frm, obj)
    return _planar_out(tr)[:, :3], _planar_out(rot)


# speedup vs baseline: 459.8341x; 1.0026x over previous
import jax
import jax.numpy as jnp
from jax import lax
from jax.experimental import pallas as pl
from jax.experimental.pallas import tpu as pltpu
from jax.experimental.pallas import tpu_sc as plsc

_C, _F, _O, _B = 6, 1000, 256, 16384
_NC, _NS, _L = 2, 16, 16
_NW = _NC * _NS
_BPW = _B // _NW
_NBT = _BPW // 128
_JPB = 128 // _L

_TD = 256000


def _pose_body(it_hbm, ir_hbm, ot_hbm, oth_hbm, cam_hbm, frm_hbm, obj_hbm,
               otr_hbm, orot_hbm,
               cam_v, frm_v, obj_v, lin_v, idxt_v, idxr_v,
               ta_v, tb_v, qr_v, th_v, otr_v, oq_v, sem, osem):
    wid = lax.axis_index("s") * _NC + lax.axis_index("c")
    base = wid * _BPW

    s1 = pltpu.async_copy(cam_hbm.at[pl.ds(base, _BPW)], cam_v, sem)
    s2 = pltpu.async_copy(frm_hbm.at[pl.ds(base, _BPW)], frm_v, sem)
    s3 = pltpu.async_copy(obj_hbm.at[pl.ds(base, _BPW)], obj_v, sem)
    s1.wait()
    s2.wait()
    s3.wait()

    for bt in range(_NBT):
        def lin_body(j, sj, bt=bt):
            s = pl.ds(bt * 128 + sj, _L)
            c16 = cam_v[s]
            f16 = frm_v[s]
            o16 = obj_v[s]
            fhi = lax.shift_right_logical(f16, 3)
            flo = lax.bitwise_and(f16, 7)
            ohi = lax.shift_right_logical(o16, 7)
            olo = lax.bitwise_and(o16, 127)
            cf = c16 * _F + f16
            p = c16 * (3 * _TD) + fhi * 2048 + ohi * 1024 + flo * 128 + olo
            q = cf * 1024 + ohi * 512 + olo
            for d in range(3):
                idxt_v[pl.ds(bt * 384 + d * 128 + sj, _L)] = p + d * _TD
            for d in range(4):
                idxr_v[pl.ds(bt * 512 + d * 128 + sj, _L)] = q + d * 128
            lin_v[s] = cf * _O + o16
            return sj + _L
        lax.fori_loop(0, _JPB, lin_body, 0)

    copies = [
        pltpu.async_copy(it_hbm.at[idxt_v], ta_v, sem),
        pltpu.async_copy(ot_hbm.at[idxt_v], tb_v, sem),
        pltpu.async_copy(ir_hbm.at[idxr_v], qr_v, sem),
        pltpu.async_copy(oth_hbm.at[lin_v], th_v, sem),
    ]
    for cp in copies:
        cp.wait()

    zero16 = jnp.zeros((_L,), jnp.float32)

    for bt in range(_NBT):
        def tr_body(j, sj, bt=bt):
            for d in range(3):
                sl = pl.ds(bt * 384 + d * 128 + sj, _L)
                otr_v[pl.ds(bt * 512 + d * 128 + sj, _L)] = ta_v[sl] + tb_v[sl]
            otr_v[pl.ds(bt * 512 + 3 * 128 + sj, _L)] = zero16
            return sj + _L
        lax.fori_loop(0, _JPB, tr_body, 0)

        def rot_body(j, carry, bt=bt):
            sth, s = carry
            th = th_v[pl.ds(sth, _L)]
            h = th * jnp.float32(0.5)
            h2 = h * h
            c = jnp.float32(1.0) + h2 * (
                jnp.float32(-1 / 2) + h2 * (
                    jnp.float32(1 / 24) + h2 * (
                        jnp.float32(-1 / 720) + h2 * jnp.float32(1 / 40320))))
            sn = h * (jnp.float32(1.0) + h2 * (
                jnp.float32(-1 / 6) + h2 * (
                    jnp.float32(1 / 120) + h2 * (
                        jnp.float32(-1 / 5040) + h2 * jnp.float32(1 / 362880)))))
            a0 = qr_v[pl.ds(s, _L)]
            a1 = qr_v[pl.ds(s + 128, _L)]
            a2 = qr_v[pl.ds(s + 2 * 128, _L)]
            a3 = qr_v[pl.ds(s + 3 * 128, _L)]
            oq_v[pl.ds(s, _L)] = a0 * c - a3 * sn
            oq_v[pl.ds(s + 128, _L)] = a1 * c + a2 * sn
            oq_v[pl.ds(s + 2 * 128, _L)] = a2 * c - a1 * sn
            oq_v[pl.ds(s + 3 * 128, _L)] = a3 * c + a0 * sn
            return (sth + _L, s + _L)
        lax.fori_loop(0, _JPB, rot_body, (bt * 128, bt * 512))

    o1 = pltpu.async_copy(otr_v, otr_hbm.at[pl.ds(wid * 2048, 2048)], osem)
    o2 = pltpu.async_copy(oq_v, orot_hbm.at[pl.ds(wid * 2048, 2048)], osem)
    o1.wait()
    o2.wait()


_pose_call = pl.kernel(
    _pose_body,
    mesh=plsc.VectorSubcoreMesh(core_axis_name="c", subcore_axis_name="s"),
    compiler_params=pltpu.CompilerParams(
        use_tc_tiling_on_sc=False, needs_layout_passes=False,
        disable_bounds_checks=True, disable_semaphore_checks=True,
        skip_device_barrier=True),
    out_type=(
        jax.ShapeDtypeStruct((_B * 4,), jnp.float32),
        jax.ShapeDtypeStruct((_B * 4,), jnp.float32),
    ),
    scratch_types=[
        pltpu.VMEM((_BPW,), jnp.int32),
        pltpu.VMEM((_BPW,), jnp.int32),
        pltpu.VMEM((_BPW,), jnp.int32),
        pltpu.VMEM((_BPW,), jnp.int32),
        pltpu.VMEM((_BPW * 3,), jnp.int32),
        pltpu.VMEM((_BPW * 4,), jnp.int32),
        pltpu.VMEM((_BPW * 3,), jnp.float32),
        pltpu.VMEM((_BPW * 3,), jnp.float32),
        pltpu.VMEM((_BPW * 4,), jnp.float32),
        pltpu.VMEM((_BPW,), jnp.float32),
        pltpu.VMEM((_BPW * 4,), jnp.float32),
        pltpu.VMEM((_BPW * 4,), jnp.float32),
        pltpu.SemaphoreType.DMA,
        pltpu.SemaphoreType.DMA,
    ],
)


def _trans_view(t):
    v = t.transpose(0, 3, 1, 2).reshape(_C, 3, _F // 8, 8, 2, 128)
    return v.transpose(0, 1, 2, 4, 3, 5).reshape(-1)


def _rots_view(t):
    return t.reshape(_C, _F, 2, 128, 4).transpose(0, 1, 2, 4, 3).reshape(-1)


def _planar_out(flat):
    return flat.reshape(_B // 128, 4, 128).transpose(0, 2, 1).reshape(_B, 4)


@jax.jit
def kernel(input_trans, input_rots, opt_trans, opt_rots, cam, frame_idx, obj_id):
    it = _trans_view(input_trans)
    ot = _trans_view(opt_trans)
    ir = _rots_view(input_rots)
    oth = opt_rots.reshape(-1)
    cam = cam.astype(jnp.int32)
    frm = frame_idx.astype(jnp.int32)
    obj = obj_id.astype(jnp.int32)
    tr, rot = _pose_call(it, ir, ot, oth, cam, frm, obj)
    return _planar_out(tr)[:, :3], _planar_out(rot)
